# baseline (device time: 66177 ns/iter reference)
import functools

import jax
import jax.numpy as jnp
from jax import lax
from jax.experimental import pallas as pl
from jax.experimental.pallas import tpu as pltpu

N_DEV = 32
N_STAGES = 5
B, SQ, DMODEL = 2, 256, 512
HQ, DH = 4, 64
NQB = SQ // 64


def kernel(x, Wq, K_ext, V_ext, Wo):
    def body(
        x_ref, wq_ref, k_ref, v_ref, wo_ref, out_ref,
        num_ref, den_ref, nrecv_ref, drecv_ref,
        nsend_sems, nrecv_sems, dsend_sems, drecv_sems,
    ):
        my = lax.axis_index("i")

        for b in range(B):
            qp = jnp.dot(
                x_ref[b], wq_ref[...], preferred_element_type=jnp.float32
            )
            for h in range(HQ):
                for qb in range(NQB):
                    qs = slice(qb * 64, (qb + 1) * 64)
                    Qb = qp[qs, h * 64:(h + 1) * 64]
                    Kb = k_ref[b, qs, h, :]
                    Vb = v_ref[b, qs, h, :]
                    sT = lax.dot_general(
                        Kb, Qb, (((1,), (1,)), ((), ())),
                        preferred_element_type=jnp.float32,
                    )
                    wT = jnp.exp(sT * 0.125)
                    numT = lax.dot_general(
                        Vb, wT, (((0,), (0,)), ((), ())),
                        preferred_element_type=jnp.float32,
                    )
                    num_ref[b, h, :, qs] = numT
                    den_ref[b * HQ + h:b * HQ + h + 1, qs] = jnp.sum(
                        wT, axis=0, keepdims=True
                    )

        partners = [jnp.bitwise_xor(my, 1 << s) for s in range(N_STAGES)]
        barrier_sem = pltpu.get_barrier_semaphore()
        for p in partners:
            pl.semaphore_signal(
                barrier_sem, inc=1, device_id=(p,),
                device_id_type=pl.DeviceIdType.MESH,
            )
        pl.semaphore_wait(barrier_sem, N_STAGES)

        for s in range(N_STAGES):
            p = partners[s]
            n_rdma = pltpu.make_async_remote_copy(
                src_ref=num_ref,
                dst_ref=nrecv_ref.at[s],
                send_sem=nsend_sems.at[s],
                recv_sem=nrecv_sems.at[s],
                device_id=(p,),
                device_id_type=pl.DeviceIdType.MESH,
            )
            d_rdma = pltpu.make_async_remote_copy(
                src_ref=den_ref,
                dst_ref=drecv_ref.at[s],
                send_sem=dsend_sems.at[s],
                recv_sem=drecv_sems.at[s],
                device_id=(p,),
                device_id_type=pl.DeviceIdType.MESH,
            )
            n_rdma.start()
            d_rdma.start()
            n_rdma.wait()
            d_rdma.wait()
            num_ref[...] = num_ref[...] + nrecv_ref[s]
            den_ref[...] = den_ref[...] + drecv_ref[s]

        for b in range(B):
            acc = jnp.zeros((SQ, DMODEL), dtype=jnp.float32)
            for h in range(HQ):
                ctxT = num_ref[b, h] / den_ref[b * HQ + h:b * HQ + h + 1, :]
                acc = acc + lax.dot_general(
                    ctxT, wo_ref[h * 64:(h + 1) * 64, :],
                    (((0,), (0,)), ((), ())),
                    preferred_element_type=jnp.float32,
                )
            out_ref[b] = acc

        @functools.partial(
            pl.run_scoped, exit_sem=pltpu.SemaphoreType.REGULAR
        )
        def _(exit_sem):
            for p in partners:
                pl.semaphore_signal(
                    exit_sem, inc=1, device_id=(p,),
                    device_id_type=pl.DeviceIdType.MESH,
                )
            pl.semaphore_wait(exit_sem, N_STAGES)

    return pl.pallas_call(
        body,
        out_shape=jax.ShapeDtypeStruct((B, SQ, DMODEL), jnp.float32),
        in_specs=[pl.BlockSpec(memory_space=pltpu.VMEM)] * 5,
        out_specs=pl.BlockSpec(memory_space=pltpu.VMEM),
        scratch_shapes=[
            pltpu.VMEM((B, HQ, DH, SQ), jnp.float32),
            pltpu.VMEM((B * HQ, SQ), jnp.float32),
            pltpu.VMEM((N_STAGES, B, HQ, DH, SQ), jnp.float32),
            pltpu.VMEM((N_STAGES, B * HQ, SQ), jnp.float32),
            pltpu.SemaphoreType.DMA((N_STAGES,)),
            pltpu.SemaphoreType.DMA((N_STAGES,)),
            pltpu.SemaphoreType.DMA((N_STAGES,)),
            pltpu.SemaphoreType.DMA((N_STAGES,)),
        ],
        compiler_params=pltpu.CompilerParams(collective_id=0),
    )(x, Wq, K_ext, V_ext, Wo)


# device time: 46637 ns/iter; 1.4190x vs baseline; 1.4190x over previous
import functools

import jax
import jax.numpy as jnp
from jax import lax
from jax.experimental import pallas as pl
from jax.experimental.pallas import tpu as pltpu

N_DEV = 32
N_STAGES = 5
B, SQ, DMODEL = 2, 256, 512
HQ, DH = 4, 64
NQB = SQ // 64


def kernel(x, Wq, K_ext, V_ext, Wo):
    def body(
        x_ref, wq_ref, k_ref, v_ref, wo_ref, out_ref,
        num_ref, den_ref, nsend_ref, nrecv_ref, drecv_ref,
        nsend_sems, nrecv_sems, dsend_sems, drecv_sems,
    ):
        my = lax.axis_index("i")

        for b in range(B):
            qp = jnp.dot(
                x_ref[b], wq_ref[...], preferred_element_type=jnp.float32
            )
            for h in range(HQ):
                for qb in range(NQB):
                    qs = slice(qb * 64, (qb + 1) * 64)
                    Qb = qp[qs, h * 64:(h + 1) * 64]
                    Kb = k_ref[b, qs, h, :]
                    Vb = v_ref[b, qs, h, :]
                    sT = lax.dot_general(
                        Kb, Qb, (((1,), (1,)), ((), ())),
                        preferred_element_type=jnp.float32,
                    )
                    wT = jnp.exp(sT * 0.125)
                    numT = lax.dot_general(
                        Vb, wT, (((0,), (0,)), ((), ())),
                        preferred_element_type=jnp.float32,
                    )
                    num_ref[b, h, :, qs] = numT
                    den_ref[b * HQ + h:b * HQ + h + 1, qs] = jnp.sum(
                        wT, axis=0, keepdims=True
                    )

        partners = [jnp.bitwise_xor(my, 1 << s) for s in range(N_STAGES)]
        barrier_sem = pltpu.get_barrier_semaphore()
        for p in partners:
            pl.semaphore_signal(
                barrier_sem, inc=1, device_id=(p,),
                device_id_type=pl.DeviceIdType.MESH,
            )
        pl.semaphore_wait(barrier_sem, N_STAGES)

        for s in range(N_STAGES):
            p = partners[s]
            nsend_ref[...] = num_ref[...].astype(jnp.bfloat16)
            n_rdma = pltpu.make_async_remote_copy(
                src_ref=nsend_ref,
                dst_ref=nrecv_ref.at[s],
                send_sem=nsend_sems.at[s],
                recv_sem=nrecv_sems.at[s],
                device_id=(p,),
                device_id_type=pl.DeviceIdType.MESH,
            )
            d_rdma = pltpu.make_async_remote_copy(
                src_ref=den_ref,
                dst_ref=drecv_ref.at[s],
                send_sem=dsend_sems.at[s],
                recv_sem=drecv_sems.at[s],
                device_id=(p,),
                device_id_type=pl.DeviceIdType.MESH,
            )
            n_rdma.start()
            d_rdma.start()
            n_rdma.wait()
            d_rdma.wait()
            num_ref[...] = num_ref[...] + nrecv_ref[s].astype(jnp.float32)
            den_ref[...] = den_ref[...] + drecv_ref[s]

        for b in range(B):
            acc = jnp.zeros((SQ, DMODEL), dtype=jnp.float32)
            for h in range(HQ):
                ctxT = num_ref[b, h] / den_ref[b * HQ + h:b * HQ + h + 1, :]
                acc = acc + lax.dot_general(
                    ctxT, wo_ref[h * 64:(h + 1) * 64, :],
                    (((0,), (0,)), ((), ())),
                    preferred_element_type=jnp.float32,
                )
            out_ref[b] = acc

        @functools.partial(
            pl.run_scoped, exit_sem=pltpu.SemaphoreType.REGULAR
        )
        def _(exit_sem):
            for p in partners:
                pl.semaphore_signal(
                    exit_sem, inc=1, device_id=(p,),
                    device_id_type=pl.DeviceIdType.MESH,
                )
            pl.semaphore_wait(exit_sem, N_STAGES)

    return pl.pallas_call(
        body,
        out_shape=jax.ShapeDtypeStruct((B, SQ, DMODEL), jnp.float32),
        in_specs=[pl.BlockSpec(memory_space=pltpu.VMEM)] * 5,
        out_specs=pl.BlockSpec(memory_space=pltpu.VMEM),
        scratch_shapes=[
            pltpu.VMEM((B, HQ, DH, SQ), jnp.float32),
            pltpu.VMEM((B * HQ, SQ), jnp.float32),
            pltpu.VMEM((B, HQ, DH, SQ), jnp.bfloat16),
            pltpu.VMEM((N_STAGES, B, HQ, DH, SQ), jnp.bfloat16),
            pltpu.VMEM((N_STAGES, B * HQ, SQ), jnp.float32),
            pltpu.SemaphoreType.DMA((N_STAGES,)),
            pltpu.SemaphoreType.DMA((N_STAGES,)),
            pltpu.SemaphoreType.DMA((N_STAGES,)),
            pltpu.SemaphoreType.DMA((N_STAGES,)),
        ],
        compiler_params=pltpu.CompilerParams(collective_id=0),
    )(x, Wq, K_ext, V_ext, Wo)


# device time: 38842 ns/iter; 1.7037x vs baseline; 1.2007x over previous
import functools

import jax
import jax.numpy as jnp
from jax import lax
from jax.experimental import pallas as pl
from jax.experimental.pallas import tpu as pltpu

N_DEV = 32
N_STAGES = 5
B, SQ, DMODEL = 2, 256, 512
HQ, DH = 4, 64
NQB = SQ // 64


def kernel(x, Wq, K_ext, V_ext, Wo):
    def body(
        x_ref, wq_ref, k_ref, v_ref, wo_ref, out_ref,
        num_ref, den_ref, nsend_ref, nrecv_ref, drecv_ref,
        nsend_sems, nrecv_sems, dsend_sems, drecv_sems,
    ):
        my = lax.axis_index("i")
        partners = [jnp.bitwise_xor(my, 1 << s) for s in range(N_STAGES)]

        def compute_partial(b):
            qp = jnp.dot(
                x_ref[b], wq_ref[...], preferred_element_type=jnp.float32
            )
            for h in range(HQ):
                for qb in range(NQB):
                    qs = slice(qb * 64, (qb + 1) * 64)
                    Qb = qp[qs, h * 64:(h + 1) * 64]
                    Kb = k_ref[b, qs, h, :]
                    Vb = v_ref[b, qs, h, :]
                    sT = lax.dot_general(
                        Kb, Qb, (((1,), (1,)), ((), ())),
                        preferred_element_type=jnp.float32,
                    )
                    wT = jnp.exp(sT * 0.125)
                    numT = lax.dot_general(
                        Vb, wT, (((0,), (0,)), ((), ())),
                        preferred_element_type=jnp.float32,
                    )
                    num_ref[b, h, :, qs] = numT
                    den_ref[b * HQ + h:b * HQ + h + 1, qs] = jnp.sum(
                        wT, axis=0, keepdims=True
                    )

        def finalize(b):
            acc = jnp.zeros((SQ, DMODEL), dtype=jnp.float32)
            for h in range(HQ):
                ctxT = num_ref[b, h] / den_ref[b * HQ + h:b * HQ + h + 1, :]
                acc = acc + lax.dot_general(
                    ctxT, wo_ref[h * 64:(h + 1) * 64, :],
                    (((0,), (0,)), ((), ())),
                    preferred_element_type=jnp.float32,
                )
            out_ref[b] = acc

        def make_n(c, s):
            return pltpu.make_async_remote_copy(
                src_ref=nsend_ref.at[c],
                dst_ref=nrecv_ref.at[s, c],
                send_sem=nsend_sems.at[s, c],
                recv_sem=nrecv_sems.at[s, c],
                device_id=(partners[s],),
                device_id_type=pl.DeviceIdType.MESH,
            )

        def make_d(s):
            return pltpu.make_async_remote_copy(
                src_ref=den_ref,
                dst_ref=drecv_ref.at[s],
                send_sem=dsend_sems.at[s],
                recv_sem=drecv_sems.at[s],
                device_id=(partners[s],),
                device_id_type=pl.DeviceIdType.MESH,
            )

        rdma_n = {}
        rdma_d = {}

        compute_partial(0)

        barrier_sem = pltpu.get_barrier_semaphore()
        for p in partners:
            pl.semaphore_signal(
                barrier_sem, inc=1, device_id=(p,),
                device_id_type=pl.DeviceIdType.MESH,
            )
        pl.semaphore_wait(barrier_sem, N_STAGES)

        nsend_ref[0] = num_ref[0].astype(jnp.bfloat16)
        rdma_n[(0, 0)] = make_n(0, 0)
        rdma_n[(0, 0)].start()
        rdma_d[0] = make_d(0)
        rdma_d[0].start()

        compute_partial(1)
        nsend_ref[1] = num_ref[1].astype(jnp.bfloat16)
        rdma_n[(1, 0)] = make_n(1, 0)
        rdma_n[(1, 0)].start()

        for s in range(N_STAGES):
            rdma_n[(0, s)].wait()
            rdma_d[s].wait()
            num_ref[0] = num_ref[0] + nrecv_ref[s, 0].astype(jnp.float32)
            den_ref[...] = den_ref[...] + drecv_ref[s]
            if s + 1 < N_STAGES:
                nsend_ref[0] = num_ref[0].astype(jnp.bfloat16)
                rdma_n[(0, s + 1)] = make_n(0, s + 1)
                rdma_n[(0, s + 1)].start()
                rdma_d[s + 1] = make_d(s + 1)
                rdma_d[s + 1].start()
            else:
                finalize(0)

            rdma_n[(1, s)].wait()
            num_ref[1] = num_ref[1] + nrecv_ref[s, 1].astype(jnp.float32)
            if s + 1 < N_STAGES:
                nsend_ref[1] = num_ref[1].astype(jnp.bfloat16)
                rdma_n[(1, s + 1)] = make_n(1, s + 1)
                rdma_n[(1, s + 1)].start()
            else:
                finalize(1)

        @functools.partial(
            pl.run_scoped, exit_sem=pltpu.SemaphoreType.REGULAR
        )
        def _(exit_sem):
            for p in partners:
                pl.semaphore_signal(
                    exit_sem, inc=1, device_id=(p,),
                    device_id_type=pl.DeviceIdType.MESH,
                )
            pl.semaphore_wait(exit_sem, N_STAGES)

    return pl.pallas_call(
        body,
        out_shape=jax.ShapeDtypeStruct((B, SQ, DMODEL), jnp.float32),
        in_specs=[pl.BlockSpec(memory_space=pltpu.VMEM)] * 5,
        out_specs=pl.BlockSpec(memory_space=pltpu.VMEM),
        scratch_shapes=[
            pltpu.VMEM((B, HQ, DH, SQ), jnp.float32),
            pltpu.VMEM((B * HQ, SQ), jnp.float32),
            pltpu.VMEM((B, HQ, DH, SQ), jnp.bfloat16),
            pltpu.VMEM((N_STAGES, B, HQ, DH, SQ), jnp.bfloat16),
            pltpu.VMEM((N_STAGES, B * HQ, SQ), jnp.float32),
            pltpu.SemaphoreType.DMA((N_STAGES, B)),
            pltpu.SemaphoreType.DMA((N_STAGES, B)),
            pltpu.SemaphoreType.DMA((N_STAGES,)),
            pltpu.SemaphoreType.DMA((N_STAGES,)),
        ],
        compiler_params=pltpu.CompilerParams(collective_id=0),
    )(x, Wq, K_ext, V_ext, Wo)
